# Initial kernel scaffold; baseline (speedup 1.0000x reference)
import functools

import jax
import jax.numpy as jnp
from jax import lax
from jax.experimental import pallas as pl
from jax.experimental.pallas import tpu as pltpu
from jax.experimental.pallas import tpu_sc as plsc

N = 10000
E = 320000
HIDDEN = 128
NUM_LAYERS = 8
G = 64
EPS = 1e-5
OUT_DIM = 64

NC = 2
NS = 16
CH = 128
NW = NC * NS
CHUNKS_W = -(-E // (NW * CH))
PER_W = CHUNKS_W * CH
E_PAD = PER_W * NW
ROWS_T = N // NS
LANES = 16


def _sc_agg_body(h_hbm, src_hbm, dst_hbm, w_hbm, out_hbm,
                 src_v, dst_v, w_v, rows_v, agg_sh, sem):
    cid = lax.axis_index("c")
    sid = lax.axis_index("s")
    wid = cid * NS + sid
    base = wid * PER_W

    pltpu.sync_copy(src_hbm.at[pl.ds(base, PER_W)], src_v)
    pltpu.sync_copy(dst_hbm.at[pl.ds(wid * CHUNKS_W, CHUNKS_W)], dst_v)
    pltpu.sync_copy(w_hbm.at[pl.ds(base, PER_W)], w_v)

    def _zrow(r, carry):
        for j in range(HIDDEN // LANES):
            rows_v[r, pl.ds(j * LANES, LANES)] = jnp.zeros((LANES,), jnp.float32)
        return carry
    lax.fori_loop(0, CH, _zrow, 0)
    for k in range(5):
        pltpu.sync_copy(rows_v.at[pl.ds(0, 125)],
                        agg_sh.at[pl.ds(sid * ROWS_T + k * 125, 125)])
    plsc.subcore_barrier()

    def _chunk(ci, carry):
        off = ci * CH
        pltpu.async_copy(h_hbm.at[src_v.at[pl.ds(off, CH)]], rows_v, sem).wait()

        def _scale(e, c2):
            wv = plsc.load_gather(w_v, [jnp.full((LANES,), off + e, jnp.int32)])
            for j in range(HIDDEN // LANES):
                sl = pl.ds(j * LANES, LANES)
                rows_v[e, sl] = rows_v[e, sl] * wv
            return c2
        lax.fori_loop(0, CH, _scale, 0)

        pltpu.sync_copy(rows_v, agg_sh.at[dst_v.at[ci]], add=True)
        return carry
    lax.fori_loop(0, CHUNKS_W, _chunk, 0)

    plsc.subcore_barrier()
    pltpu.sync_copy(agg_sh.at[pl.ds(sid * ROWS_T, ROWS_T)],
                    out_hbm.at[pl.ds(cid * N + sid * ROWS_T, ROWS_T)])


_sc_agg = functools.partial(
    pl.kernel,
    out_type=jax.ShapeDtypeStruct((NC * N, HIDDEN), jnp.float32),
    mesh=plsc.VectorSubcoreMesh(core_axis_name="c", subcore_axis_name="s"),
    scratch_types=[
        pltpu.VMEM((PER_W,), jnp.int32),
        pltpu.VMEM((CHUNKS_W, CH), jnp.int32),
        pltpu.VMEM((PER_W,), jnp.float32),
        pltpu.VMEM((CH, HIDDEN), jnp.float32),
        pltpu.VMEM_SHARED((N, HIDDEN), jnp.float32),
        pltpu.SemaphoreType.DMA,
    ],
)(_sc_agg_body)


def _tc_update_body(h_ref, agg_ref, w0_ref, w1_ref, b0_ref, b1_ref,
                    gam_ref, bet_ref, hn_ref):
    m = h_ref[...] + agg_ref[0] + agg_ref[1]
    t = jnp.dot(m, w0_ref[...], preferred_element_type=jnp.float32) + b0_ref[...]
    t = jnp.maximum(t, 0.0)
    t = jnp.dot(t, w1_ref[...], preferred_element_type=jnp.float32) + b1_ref[...]
    hn = jnp.maximum(t, 0.0)
    mean = jnp.mean(hn, axis=0, keepdims=True)
    var = jnp.mean((hn - mean) ** 2, axis=0, keepdims=True)
    hn_ref[...] = (hn - mean) / jnp.sqrt(var + EPS) * gam_ref[...] + bet_ref[...]


_tc_update = pl.pallas_call(
    _tc_update_body,
    out_shape=jax.ShapeDtypeStruct((N, HIDDEN), jnp.float32),
)


def _tc_readout_body(h_ref, gw_ref, gb_ref, aw_ref, ab_ref,
                     bcol_ref, brow_ref, out_ref):
    hn = h_ref[...]
    gate = jnp.sum(hn * gw_ref[...], axis=1, keepdims=True) + gb_ref[...]
    bcol = bcol_ref[...]
    seg = lax.broadcasted_iota(jnp.int32, (N, G), 1)
    mask = bcol == seg
    gmax = jnp.max(jnp.where(mask, gate, -1e30), axis=0, keepdims=True)
    gmax_b = jnp.sum(jnp.where(mask, gmax, 0.0), axis=1, keepdims=True)
    e = jnp.exp(gate - gmax_b)
    denom = jnp.sum(jnp.where(mask, e, 0.0), axis=0, keepdims=True)
    denom_b = jnp.sum(jnp.where(mask, denom, 0.0), axis=1, keepdims=True)
    alpha = e / (denom_b + 1e-16)
    v = jnp.dot(hn, aw_ref[...], preferred_element_type=jnp.float32) + ab_ref[...]
    av = alpha * v
    brow = brow_ref[...]
    segt = lax.broadcasted_iota(jnp.int32, (G, N), 0)
    maskt = (segt == brow).astype(jnp.float32)
    out_ref[...] = jnp.dot(maskt, av, preferred_element_type=jnp.float32)


_tc_readout = pl.pallas_call(
    _tc_readout_body,
    out_shape=jax.ShapeDtypeStruct((G, HIDDEN), jnp.float32),
)


def _tc_head_body(c_ref, w1_ref, b1_ref, w2_ref, b2_ref, o_ref):
    hid = jnp.dot(c_ref[...], w1_ref[...], preferred_element_type=jnp.float32)
    hid = jnp.maximum(hid + b1_ref[...], 0.0)
    o_ref[...] = jnp.dot(hid, w2_ref[...],
                         preferred_element_type=jnp.float32) + b2_ref[...]


_tc_head = pl.pallas_call(
    _tc_head_body,
    out_shape=jax.ShapeDtypeStruct((G, OUT_DIM), jnp.float32),
)


def kernel(x, edge_weight, W_conv, b_conv, bn_gamma, bn_beta, gate_w, gate_b,
           att_w, att_b, pred_w1, pred_b1, pred_w2, pred_b2, edge_index, batch):
    h = jnp.pad(x.reshape(-1, 1), ((0, 0), (0, HIDDEN - 1)))
    src = edge_index[0].astype(jnp.int32)
    dst = edge_index[1].astype(jnp.int32)
    pad = E_PAD - E
    zi = jnp.zeros((pad,), jnp.int32)
    srcp = jnp.concatenate([src, zi])
    dstp = jnp.concatenate([dst, zi]).reshape(NW * CHUNKS_W, CH)
    wp = jnp.concatenate([edge_weight, jnp.zeros((pad,), jnp.float32)])
    bcol = batch.reshape(N, 1).astype(jnp.int32)
    brow = batch.reshape(1, N).astype(jnp.int32)
    gw_row = gate_w.reshape(1, HIDDEN)
    gb = gate_b.reshape(1, 1)
    ab_row = att_b.reshape(1, HIDDEN)

    outs = []
    for i in range(NUM_LAYERS):
        agg2 = _sc_agg(h, srcp, dstp, wp).reshape(NC, N, HIDDEN)
        h = _tc_update(h, agg2, W_conv[i, 0], W_conv[i, 1],
                       b_conv[i, 0].reshape(1, HIDDEN),
                       b_conv[i, 1].reshape(1, HIDDEN),
                       bn_gamma[i].reshape(1, HIDDEN),
                       bn_beta[i].reshape(1, HIDDEN))
        outs.append(_tc_readout(h, gw_row, gb, att_w, ab_row, bcol, brow))
    cat = jnp.concatenate(outs, axis=1)
    return _tc_head(cat, pred_w1, pred_b1.reshape(1, OUT_DIM),
                    pred_w2, pred_b2.reshape(1, OUT_DIM))


# speedup vs baseline: 1.1040x; 1.1040x over previous
import jax
import jax.numpy as jnp
from jax import lax
from jax.experimental import pallas as pl

N = 10000
E = 320000
HIDDEN = 128
NUM_LAYERS = 8
MLP_LAYERS = 2
G = 64
EPS = 1e-5
OUT_DIM = 64


def _tc_readout_body(h_ref, gw_ref, gb_ref, aw_ref, ab_ref,
                     bcol_ref, brow_ref, out_ref):
    hn = h_ref[...]
    gate = jnp.dot(hn, gw_ref[...],
                   preferred_element_type=jnp.float32) + gb_ref[...]
    bcol = bcol_ref[...]
    seg = lax.broadcasted_iota(jnp.int32, (N, G), 1)
    mask = bcol == seg
    gmax = jnp.max(jnp.where(mask, gate, -1e30), axis=0, keepdims=True)
    gmax_b = jnp.sum(jnp.where(mask, gmax, 0.0), axis=1, keepdims=True)
    e = jnp.exp(gate - gmax_b)
    denom = jnp.sum(jnp.where(mask, e, 0.0), axis=0, keepdims=True)
    denom_b = jnp.sum(jnp.where(mask, denom, 0.0), axis=1, keepdims=True)
    alpha = e / (denom_b + 1e-16)
    v = jnp.dot(hn, aw_ref[...], preferred_element_type=jnp.float32) + ab_ref[...]
    av = alpha * v
    brow = brow_ref[...]
    segt = lax.broadcasted_iota(jnp.int32, (G, N), 0)
    maskt = (segt == brow).astype(jnp.float32)
    out_ref[...] = jnp.dot(maskt, av, preferred_element_type=jnp.float32,
                           precision=lax.Precision.HIGHEST)


_tc_readout = pl.pallas_call(
    _tc_readout_body,
    out_shape=jax.ShapeDtypeStruct((G, HIDDEN), jnp.float32),
)


def _tc_head_body(c_ref, w1_ref, b1_ref, w2_ref, b2_ref, o_ref):
    hid = jnp.dot(c_ref[...], w1_ref[...], preferred_element_type=jnp.float32)
    hid = jnp.maximum(hid + b1_ref[...], 0.0)
    o_ref[...] = jnp.dot(hid, w2_ref[...],
                         preferred_element_type=jnp.float32) + b2_ref[...]


_tc_head = pl.pallas_call(
    _tc_head_body,
    out_shape=jax.ShapeDtypeStruct((G, OUT_DIM), jnp.float32),
)


def kernel(x, edge_weight, W_conv, b_conv, bn_gamma, bn_beta, gate_w, gate_b,
           att_w, att_b, pred_w1, pred_b1, pred_w2, pred_b2, edge_index, batch):
    src = edge_index[0]
    dst = edge_index[1]
    h = jnp.pad(x.reshape(-1, 1), ((0, 0), (0, HIDDEN - 1)))
    w = edge_weight.reshape(-1, 1)
    bcol = batch.reshape(N, 1).astype(jnp.int32)
    brow = batch.reshape(1, N).astype(jnp.int32)
    gb = gate_b.reshape(1, 1)
    ab_row = att_b.reshape(1, HIDDEN)
    outs = []
    for i in range(NUM_LAYERS):
        msg = h[src] * w
        agg = jax.ops.segment_sum(msg, dst, num_segments=N)
        m = h + agg
        for l in range(MLP_LAYERS):
            m = m @ W_conv[i, l] + b_conv[i, l]
            if l < MLP_LAYERS - 1:
                m = jax.nn.relu(m)
        h = jax.nn.relu(m)
        mean = jnp.mean(h, axis=0)
        var = jnp.mean((h - mean) ** 2, axis=0)
        h = (h - mean) / jnp.sqrt(var + EPS) * bn_gamma[i] + bn_beta[i]
        outs.append(_tc_readout(h, gate_w, gb, att_w, ab_row, bcol, brow))
    cat_h = jnp.concatenate(outs, axis=1)
    return _tc_head(cat_h, pred_w1, pred_b1.reshape(1, OUT_DIM),
                    pred_w2, pred_b2.reshape(1, OUT_DIM))


# speedup vs baseline: 1.1044x; 1.0003x over previous
import jax
import jax.numpy as jnp
from jax import lax
from jax.experimental import pallas as pl

N = 10000
E = 320000
HIDDEN = 128
NUM_LAYERS = 8
MLP_LAYERS = 2
G = 64
EPS = 1e-5
OUT_DIM = 64


def _tc_readout_body(h_ref, gw_ref, gb_ref, aw_ref, ab_ref,
                     bcol_ref, brow_ref, out_ref):
    hn = h_ref[...]
    gate = jnp.dot(hn, gw_ref[...],
                   preferred_element_type=jnp.float32) + gb_ref[...]
    bcol = bcol_ref[...]
    seg = lax.broadcasted_iota(jnp.int32, (N, G), 1)
    mask = bcol == seg
    gmax = jnp.max(jnp.where(mask, gate, -1e30), axis=0, keepdims=True)
    gmax_b = jnp.sum(jnp.where(mask, gmax, 0.0), axis=1, keepdims=True)
    e = jnp.exp(gate - gmax_b)
    denom = jnp.sum(jnp.where(mask, e, 0.0), axis=0, keepdims=True)
    denom_b = jnp.sum(jnp.where(mask, denom, 0.0), axis=1, keepdims=True)
    alpha = e / (denom_b + 1e-16)
    v = jnp.dot(hn, aw_ref[...], preferred_element_type=jnp.float32) + ab_ref[...]
    av = alpha * v
    brow = brow_ref[...]
    segt = lax.broadcasted_iota(jnp.int32, (G, N), 0)
    maskt = (segt == brow).astype(jnp.float32)
    av_hi = av.astype(jnp.bfloat16).astype(jnp.float32)
    av_lo = av - av_hi
    out_ref[...] = (
        jnp.dot(maskt, av_hi, preferred_element_type=jnp.float32)
        + jnp.dot(maskt, av_lo, preferred_element_type=jnp.float32))


_tc_readout = pl.pallas_call(
    _tc_readout_body,
    out_shape=jax.ShapeDtypeStruct((G, HIDDEN), jnp.float32),
)


def _tc_head_body(c_ref, w1_ref, b1_ref, w2_ref, b2_ref, o_ref):
    hid = jnp.dot(c_ref[...], w1_ref[...], preferred_element_type=jnp.float32)
    hid = jnp.maximum(hid + b1_ref[...], 0.0)
    o_ref[...] = jnp.dot(hid, w2_ref[...],
                         preferred_element_type=jnp.float32) + b2_ref[...]


_tc_head = pl.pallas_call(
    _tc_head_body,
    out_shape=jax.ShapeDtypeStruct((G, OUT_DIM), jnp.float32),
)


def kernel(x, edge_weight, W_conv, b_conv, bn_gamma, bn_beta, gate_w, gate_b,
           att_w, att_b, pred_w1, pred_b1, pred_w2, pred_b2, edge_index, batch):
    src = edge_index[0]
    dst = edge_index[1]
    h = jnp.pad(x.reshape(-1, 1), ((0, 0), (0, HIDDEN - 1)))
    w = edge_weight.reshape(-1, 1)
    bcol = batch.reshape(N, 1).astype(jnp.int32)
    brow = batch.reshape(1, N).astype(jnp.int32)
    gb = gate_b.reshape(1, 1)
    ab_row = att_b.reshape(1, HIDDEN)
    outs = []
    for i in range(NUM_LAYERS):
        msg = h[src] * w
        agg = jax.ops.segment_sum(msg, dst, num_segments=N)
        m = h + agg
        for l in range(MLP_LAYERS):
            m = m @ W_conv[i, l] + b_conv[i, l]
            if l < MLP_LAYERS - 1:
                m = jax.nn.relu(m)
        h = jax.nn.relu(m)
        mean = jnp.mean(h, axis=0)
        var = jnp.mean((h - mean) ** 2, axis=0)
        h = (h - mean) / jnp.sqrt(var + EPS) * bn_gamma[i] + bn_beta[i]
        outs.append(_tc_readout(h, gate_w, gb, att_w, ab_row, bcol, brow))
    cat_h = jnp.concatenate(outs, axis=1)
    return _tc_head(cat_h, pred_w1, pred_b1.reshape(1, OUT_DIM),
                    pred_w2, pred_b2.reshape(1, OUT_DIM))
